# Initial kernel scaffold; baseline (speedup 1.0000x reference)
import jax
import jax.numpy as jnp
from jax.experimental import pallas as pl


def kernel(lattice, fracs, species, batch_indices, emb, W1, b1, W2, b2, W3, b3, W4, b4):
    raise NotImplementedError("write your pallas kernel here")


# speedup vs baseline: 1.6411x; 1.6411x over previous
import functools

import jax
import jax.numpy as jnp
from jax import lax
from jax.experimental import pallas as pl
from jax.experimental.pallas import tpu as pltpu
from jax.experimental.pallas import tpu_sc as plsc

G = 10000
N = 320000
NC = 2
NS = 16
NW = NC * NS

B1 = 640
PER_W = N // NW
CHUNK = 80
SW = 256
GPAD = 10240
BG = 1000


def _atom_mlp_body(sp_ref, fr_ref, emb_ref, w1e_ref, w1f_ref, b1_ref,
                   w2_ref, b2_ref, out_ref):
    sp = sp_ref[...]
    cols = lax.broadcasted_iota(jnp.int32, (B1, 128), 1)
    onehot = (cols == sp).astype(jnp.float32)
    e = jnp.dot(onehot, emb_ref[...], preferred_element_type=jnp.float32)
    h = (jnp.dot(e, w1e_ref[...], preferred_element_type=jnp.float32)
         + jnp.dot(fr_ref[...], w1f_ref[...], preferred_element_type=jnp.float32)
         + b1_ref[...])
    h = jnp.maximum(h, 0.0)
    out_ref[...] = (jnp.dot(h, w2_ref[...], preferred_element_type=jnp.float32)
                    + b2_ref[...])


def _atom_mlp(species2d, fracs, emb_p, w1e_t, w1f_t, b1r, w2_t, b2r):
    grid = N // B1
    return pl.pallas_call(
        _atom_mlp_body,
        grid=(grid,),
        in_specs=[
            pl.BlockSpec((B1, 1), lambda i: (i, 0)),
            pl.BlockSpec((B1, 3), lambda i: (i, 0)),
            pl.BlockSpec((128, 32), lambda i: (0, 0)),
            pl.BlockSpec((32, 128), lambda i: (0, 0)),
            pl.BlockSpec((3, 128), lambda i: (0, 0)),
            pl.BlockSpec((1, 128), lambda i: (0, 0)),
            pl.BlockSpec((128, 128), lambda i: (0, 0)),
            pl.BlockSpec((1, 128), lambda i: (0, 0)),
        ],
        out_specs=pl.BlockSpec((B1, 128), lambda i: (i, 0)),
        out_shape=jax.ShapeDtypeStruct((N, 128), jnp.float32),
        compiler_params=pltpu.CompilerParams(
            dimension_semantics=("parallel",)),
    )(species2d, fracs, emb_p, w1e_t, w1f_t, b1r, w2_t, b2r)


def _bincount_body(idx_ref, out_ref):
    b = pl.program_id(0)

    @pl.when(b == 0)
    def _zero():
        out_ref[...] = jnp.zeros_like(out_ref)

    iv = idx_ref[...]
    lo = idx_ref[0, 0]
    hi = idx_ref[B1 - 1, 0]
    base = (lo // SW) * SW
    n_strips = (hi - base) // SW + 1

    def strip(k, _):
        off = base + k * SW
        cols = off + lax.broadcasted_iota(jnp.int32, (B1, SW), 1)
        m = (cols == iv).astype(jnp.float32)
        out_ref[:, pl.ds(off, SW)] += jnp.sum(m, axis=0, keepdims=True)
        return 0

    lax.fori_loop(0, n_strips, strip, 0)


def _bincount(idx_col):
    grid = N // B1
    return pl.pallas_call(
        _bincount_body,
        grid=(grid,),
        in_specs=[pl.BlockSpec((B1, 1), lambda i: (i, 0))],
        out_specs=pl.BlockSpec((1, GPAD), lambda i: (0, 0)),
        out_shape=jax.ShapeDtypeStruct((1, GPAD), jnp.float32),
        compiler_params=pltpu.CompilerParams(
            dimension_semantics=("arbitrary",)),
    )(idx_col)


def _seg_sum_sc(atom_feat, idx1d, zeros_p):
    mesh = plsc.VectorSubcoreMesh(core_axis_name="c", subcore_axis_name="s")
    n_iter = PER_W // CHUNK

    @functools.partial(
        pl.kernel, mesh=mesh,
        out_type=[jax.ShapeDtypeStruct((NC, G, 128), jnp.float32)],
        scratch_types=[
            pltpu.VMEM((CHUNK, 128), jnp.float32),
            pltpu.VMEM((CHUNK,), jnp.int32),
            pltpu.VMEM_SHARED((G, 128), jnp.float32),
        ],
    )
    def k(af_hbm, idx_hbm, zp_hbm, pp_hbm, rows_v, idx_v, shared_p):
        c = lax.axis_index("c")
        s = lax.axis_index("s")
        wid = c * NS + s

        @pl.when(s == 0)
        def _init():
            pltpu.sync_copy(zp_hbm, shared_p)

        plsc.subcore_barrier()

        def body(i, _):
            rowbase = wid * PER_W + i * CHUNK
            pltpu.sync_copy(idx_hbm.at[pl.ds(rowbase, CHUNK)], idx_v)
            pltpu.sync_copy(af_hbm.at[pl.ds(rowbase, CHUNK)], rows_v)
            pltpu.sync_copy(rows_v, shared_p.at[idx_v], add=True)
            return 0

        lax.fori_loop(0, n_iter, body, 0)
        plsc.subcore_barrier()

        @pl.when(s == 0)
        def _flush():
            pltpu.sync_copy(shared_p, pp_hbm.at[c])

    return k(atom_feat, idx1d, zeros_p)[0]


def _graph_mlp_body(pp_ref, cc_ref, lat_ref, w3p_ref, w3l_ref, b3_ref,
                    w4_ref, b4_ref, mu_ref, lv_ref):
    pooled = pp_ref[0] + pp_ref[1]
    pooled = pooled / cc_ref[...]
    h2 = (jnp.dot(pooled, w3p_ref[...], preferred_element_type=jnp.float32)
          + jnp.dot(lat_ref[...], w3l_ref[...], preferred_element_type=jnp.float32)
          + b3_ref[...])
    h2 = jnp.maximum(h2, 0.0)
    params = (jnp.dot(h2, w4_ref[...], preferred_element_type=jnp.float32)
              + b4_ref[...])
    mu_ref[...] = params[:, :128]
    lv_ref[...] = params[:, 128:]


def _graph_mlp(pp, cc, lat9, w3p_t, w3l_t, b3r, w4_t, b4r):
    grid = G // BG
    return pl.pallas_call(
        _graph_mlp_body,
        grid=(grid,),
        in_specs=[
            pl.BlockSpec((NC, BG, 128), lambda i: (0, i, 0)),
            pl.BlockSpec((BG, 1), lambda i: (i, 0)),
            pl.BlockSpec((BG, 9), lambda i: (i, 0)),
            pl.BlockSpec((128, 128), lambda i: (0, 0)),
            pl.BlockSpec((9, 128), lambda i: (0, 0)),
            pl.BlockSpec((1, 128), lambda i: (0, 0)),
            pl.BlockSpec((128, 256), lambda i: (0, 0)),
            pl.BlockSpec((1, 256), lambda i: (0, 0)),
        ],
        out_specs=[pl.BlockSpec((BG, 128), lambda i: (i, 0)),
                   pl.BlockSpec((BG, 128), lambda i: (i, 0))],
        out_shape=[jax.ShapeDtypeStruct((G, 128), jnp.float32),
                   jax.ShapeDtypeStruct((G, 128), jnp.float32)],
        compiler_params=pltpu.CompilerParams(
            dimension_semantics=("parallel",)),
    )(pp, cc, lat9, w3p_t, w3l_t, b3r, w4_t, b4r)


def kernel(lattice, fracs, species, batch_indices, emb, W1, b1, W2, b2,
           W3, b3, W4, b4):
    idx1d = batch_indices.astype(jnp.int32)
    species2d = species.astype(jnp.int32).reshape(N, 1)

    emb_p = jnp.zeros((128, 32), jnp.float32).at[:100].set(emb)
    w1e_t = W1[:, :32].T
    w1f_t = W1[:, 32:].T
    w2_t = W2.T
    w3p_t = W3[:, :128].T
    w3l_t = W3[:, 128:].T
    w4_t = W4.T

    atom_feat = _atom_mlp(species2d, fracs, emb_p, w1e_t, w1f_t,
                          b1.reshape(1, 128), w2_t, b2.reshape(1, 128))

    counts = _bincount(idx1d.reshape(N, 1))
    cc = counts[0, :G].reshape(G, 1)

    zeros_p = jnp.zeros((G, 128), jnp.float32)
    pp = _seg_sum_sc(atom_feat, idx1d, zeros_p)

    lat9 = lattice.reshape(G, 9)
    mu, logvar = _graph_mlp(pp, cc, lat9, w3p_t, w3l_t,
                            b3.reshape(1, 128), w4_t, b4.reshape(1, 256))
    return (mu, logvar)


# speedup vs baseline: 2.1515x; 1.3110x over previous
import functools

import jax
import jax.numpy as jnp
from jax import lax
from jax.experimental import pallas as pl
from jax.experimental.pallas import tpu as pltpu
from jax.experimental.pallas import tpu_sc as plsc

G = 10000
N = 320000
NC = 2
NS = 16
NW = NC * NS

B1 = 3200
PER_W = N // NW
CHUNK = 80
SW = 512
GPAD = 10240
BG = 2000


def _atom_mlp_body(sp_ref, fr_ref, idx_ref, emb_ref, w1e_ref, w1f_ref, b1_ref,
                   w2_ref, b2_ref, out_ref, cnt_ref):
    b = pl.program_id(0)

    @pl.when(b == 0)
    def _zero():
        cnt_ref[...] = jnp.zeros_like(cnt_ref)

    sp = sp_ref[...]
    cols = lax.broadcasted_iota(jnp.int32, (B1, 128), 1)
    onehot = (cols == sp).astype(jnp.float32)
    e = jnp.dot(onehot, emb_ref[...], preferred_element_type=jnp.float32)
    h = (jnp.dot(e, w1e_ref[...], preferred_element_type=jnp.float32)
         + jnp.dot(fr_ref[...], w1f_ref[...], preferred_element_type=jnp.float32)
         + b1_ref[...])
    h = jnp.maximum(h, 0.0)
    out_ref[...] = (jnp.dot(h, w2_ref[...], preferred_element_type=jnp.float32)
                    + b2_ref[...])

    iv = idx_ref[...]
    lo = idx_ref[0, 0]
    hi = idx_ref[B1 - 1, 0]
    base = (lo // SW) * SW
    n_strips = (hi - base) // SW + 1

    def strip(k, _):
        off = base + k * SW
        ccols = off + lax.broadcasted_iota(jnp.int32, (B1, SW), 1)
        m = (ccols == iv).astype(jnp.float32)
        cnt_ref[:, pl.ds(off, SW)] += jnp.sum(m, axis=0, keepdims=True)
        return 0

    lax.fori_loop(0, n_strips, strip, 0)


def _atom_mlp(species2d, fracs, idx_col, emb_p, w1e_t, w1f_t, b1r, w2_t, b2r):
    grid = N // B1
    return pl.pallas_call(
        _atom_mlp_body,
        grid=(grid,),
        in_specs=[
            pl.BlockSpec((B1, 1), lambda i: (i, 0)),
            pl.BlockSpec((B1, 3), lambda i: (i, 0)),
            pl.BlockSpec((B1, 1), lambda i: (i, 0)),
            pl.BlockSpec((128, 32), lambda i: (0, 0)),
            pl.BlockSpec((32, 128), lambda i: (0, 0)),
            pl.BlockSpec((3, 128), lambda i: (0, 0)),
            pl.BlockSpec((1, 128), lambda i: (0, 0)),
            pl.BlockSpec((128, 128), lambda i: (0, 0)),
            pl.BlockSpec((1, 128), lambda i: (0, 0)),
        ],
        out_specs=[pl.BlockSpec((B1, 128), lambda i: (i, 0)),
                   pl.BlockSpec((1, GPAD), lambda i: (0, 0))],
        out_shape=[jax.ShapeDtypeStruct((N, 128), jnp.float32),
                   jax.ShapeDtypeStruct((1, GPAD), jnp.float32)],
        compiler_params=pltpu.CompilerParams(
            dimension_semantics=("arbitrary",)),
    )(species2d, fracs, idx_col, emb_p, w1e_t, w1f_t, b1r, w2_t, b2r)


def _seg_sum_sc(atom_feat, idx1d, zeros_p):
    mesh = plsc.VectorSubcoreMesh(core_axis_name="c", subcore_axis_name="s")
    n_iter = PER_W // CHUNK

    @functools.partial(
        pl.kernel, mesh=mesh,
        out_type=[jax.ShapeDtypeStruct((NC, G, 128), jnp.float32)],
        scratch_types=[
            pltpu.VMEM((CHUNK, 128), jnp.float32),
            pltpu.VMEM((CHUNK,), jnp.int32),
            pltpu.VMEM_SHARED((G, 128), jnp.float32),
        ],
    )
    def k(af_hbm, idx_hbm, zp_hbm, pp_hbm, rows_v, idx_v, shared_p):
        c = lax.axis_index("c")
        s = lax.axis_index("s")
        wid = c * NS + s

        @pl.when(s == 0)
        def _init():
            pltpu.sync_copy(zp_hbm, shared_p)

        plsc.subcore_barrier()

        def body(i, _):
            rowbase = wid * PER_W + i * CHUNK
            pltpu.sync_copy(idx_hbm.at[pl.ds(rowbase, CHUNK)], idx_v)
            pltpu.sync_copy(af_hbm.at[pl.ds(rowbase, CHUNK)], rows_v)
            pltpu.sync_copy(rows_v, shared_p.at[idx_v], add=True)
            return 0

        lax.fori_loop(0, n_iter, body, 0)
        plsc.subcore_barrier()

        @pl.when(s == 0)
        def _flush():
            pltpu.sync_copy(shared_p, pp_hbm.at[c])

    return k(atom_feat, idx1d, zeros_p)[0]


def _graph_mlp_body(pp_ref, cc_ref, lat_ref, w3p_ref, w3l_ref, b3_ref,
                    w4_ref, b4_ref, mu_ref, lv_ref):
    pooled = pp_ref[0] + pp_ref[1]
    pooled = pooled / cc_ref[...]
    h2 = (jnp.dot(pooled, w3p_ref[...], preferred_element_type=jnp.float32)
          + jnp.dot(lat_ref[...], w3l_ref[...], preferred_element_type=jnp.float32)
          + b3_ref[...])
    h2 = jnp.maximum(h2, 0.0)
    params = (jnp.dot(h2, w4_ref[...], preferred_element_type=jnp.float32)
              + b4_ref[...])
    mu_ref[...] = params[:, :128]
    lv_ref[...] = params[:, 128:]


def _graph_mlp(pp, cc, lat9, w3p_t, w3l_t, b3r, w4_t, b4r):
    grid = G // BG
    return pl.pallas_call(
        _graph_mlp_body,
        grid=(grid,),
        in_specs=[
            pl.BlockSpec((NC, BG, 128), lambda i: (0, i, 0)),
            pl.BlockSpec((BG, 1), lambda i: (i, 0)),
            pl.BlockSpec((BG, 9), lambda i: (i, 0)),
            pl.BlockSpec((128, 128), lambda i: (0, 0)),
            pl.BlockSpec((9, 128), lambda i: (0, 0)),
            pl.BlockSpec((1, 128), lambda i: (0, 0)),
            pl.BlockSpec((128, 256), lambda i: (0, 0)),
            pl.BlockSpec((1, 256), lambda i: (0, 0)),
        ],
        out_specs=[pl.BlockSpec((BG, 128), lambda i: (i, 0)),
                   pl.BlockSpec((BG, 128), lambda i: (i, 0))],
        out_shape=[jax.ShapeDtypeStruct((G, 128), jnp.float32),
                   jax.ShapeDtypeStruct((G, 128), jnp.float32)],
        compiler_params=pltpu.CompilerParams(
            dimension_semantics=("parallel",)),
    )(pp, cc, lat9, w3p_t, w3l_t, b3r, w4_t, b4r)


def kernel(lattice, fracs, species, batch_indices, emb, W1, b1, W2, b2,
           W3, b3, W4, b4):
    idx1d = batch_indices.astype(jnp.int32)
    species2d = species.astype(jnp.int32).reshape(N, 1)

    emb_p = jnp.zeros((128, 32), jnp.float32).at[:100].set(emb)
    w1e_t = W1[:, :32].T
    w1f_t = W1[:, 32:].T
    w2_t = W2.T
    w3p_t = W3[:, :128].T
    w3l_t = W3[:, 128:].T
    w4_t = W4.T

    atom_feat, counts = _atom_mlp(species2d, fracs, idx1d.reshape(N, 1),
                                  emb_p, w1e_t, w1f_t,
                                  b1.reshape(1, 128), w2_t, b2.reshape(1, 128))
    cc = counts[0, :G].reshape(G, 1)

    zeros_p = jnp.zeros((G, 128), jnp.float32)
    pp = _seg_sum_sc(atom_feat, idx1d, zeros_p)

    lat9 = lattice.reshape(G, 9)
    mu, logvar = _graph_mlp(pp, cc, lat9, w3p_t, w3l_t,
                            b3.reshape(1, 128), w4_t, b4.reshape(1, 256))
    return (mu, logvar)


# speedup vs baseline: 3.2273x; 1.5000x over previous
import functools

import jax
import jax.numpy as jnp
from jax import lax
from jax.experimental import pallas as pl
from jax.experimental.pallas import tpu as pltpu
from jax.experimental.pallas import tpu_sc as plsc

G = 10000
N = 320000
NC = 2
NS = 16
NW = NC * NS

B1 = 3200
PER_W = N // NW
CHUNK = 80
SW = 512
GPAD = 10240
BG = 2000


def _atom_mlp_body(sp_ref, fr_ref, idx_ref, emb_ref, w1e_ref, w1f_ref, b1_ref,
                   w2_ref, b2_ref, out_ref, cnt_ref):
    b = pl.program_id(0)

    @pl.when(b == 0)
    def _zero():
        cnt_ref[...] = jnp.zeros_like(cnt_ref)

    sp = sp_ref[...]
    rows = lax.broadcasted_iota(jnp.int32, (128, B1), 0)
    onehot_t = (rows == sp).astype(jnp.float32)
    e = lax.dot_general(onehot_t, emb_ref[...], (((0,), (0,)), ((), ())),
                        preferred_element_type=jnp.float32)
    h = (jnp.dot(e, w1e_ref[...], preferred_element_type=jnp.float32)
         + jnp.dot(fr_ref[...], w1f_ref[...], preferred_element_type=jnp.float32)
         + b1_ref[...])
    h = jnp.maximum(h, 0.0)
    out_ref[...] = (jnp.dot(h, w2_ref[...], preferred_element_type=jnp.float32)
                    + b2_ref[...])

    iv = idx_ref[...]
    lo = idx_ref[0, 0]
    hi = idx_ref[0, B1 - 1]
    base = (lo // SW) * SW
    n_strips = (hi - base) // SW + 1

    def strip(k, _):
        off = base + k * SW
        srows = off + lax.broadcasted_iota(jnp.int32, (SW, B1), 0)
        m = (srows == iv).astype(jnp.float32)
        cnt_ref[pl.ds(off, SW), :] += jnp.sum(m, axis=1, keepdims=True)
        return 0

    lax.fori_loop(0, n_strips, strip, 0)


def _atom_mlp(species_row, fracs, idx_row, emb_p, w1e_t, w1f_t, b1r, w2_t, b2r):
    grid = N // B1
    return pl.pallas_call(
        _atom_mlp_body,
        grid=(grid,),
        in_specs=[
            pl.BlockSpec((1, B1), lambda i: (0, i)),
            pl.BlockSpec((B1, 3), lambda i: (i, 0)),
            pl.BlockSpec((1, B1), lambda i: (0, i)),
            pl.BlockSpec((128, 32), lambda i: (0, 0)),
            pl.BlockSpec((32, 128), lambda i: (0, 0)),
            pl.BlockSpec((3, 128), lambda i: (0, 0)),
            pl.BlockSpec((1, 128), lambda i: (0, 0)),
            pl.BlockSpec((128, 128), lambda i: (0, 0)),
            pl.BlockSpec((1, 128), lambda i: (0, 0)),
        ],
        out_specs=[pl.BlockSpec((B1, 128), lambda i: (i, 0)),
                   pl.BlockSpec((GPAD, 1), lambda i: (0, 0))],
        out_shape=[jax.ShapeDtypeStruct((N, 128), jnp.float32),
                   jax.ShapeDtypeStruct((GPAD, 1), jnp.float32)],
        compiler_params=pltpu.CompilerParams(
            dimension_semantics=("arbitrary",)),
    )(species_row, fracs, idx_row, emb_p, w1e_t, w1f_t, b1r, w2_t, b2r)


def _seg_sum_sc(atom_feat, idx1d, zeros_p):
    mesh = plsc.VectorSubcoreMesh(core_axis_name="c", subcore_axis_name="s")
    n_iter = PER_W // CHUNK

    @functools.partial(
        pl.kernel, mesh=mesh,
        out_type=[jax.ShapeDtypeStruct((NC, G, 128), jnp.float32)],
        scratch_types=[
            pltpu.VMEM((CHUNK, 128), jnp.float32),
            pltpu.VMEM((CHUNK,), jnp.int32),
            pltpu.VMEM_SHARED((G, 128), jnp.float32),
        ],
    )
    def k(af_hbm, idx_hbm, zp_hbm, pp_hbm, rows_v, idx_v, shared_p):
        c = lax.axis_index("c")
        s = lax.axis_index("s")
        wid = c * NS + s

        @pl.when(s == 0)
        def _init():
            pltpu.sync_copy(zp_hbm, shared_p)

        plsc.subcore_barrier()

        def body(i, _):
            rowbase = wid * PER_W + i * CHUNK
            pltpu.sync_copy(idx_hbm.at[pl.ds(rowbase, CHUNK)], idx_v)
            pltpu.sync_copy(af_hbm.at[pl.ds(rowbase, CHUNK)], rows_v)
            pltpu.sync_copy(rows_v, shared_p.at[idx_v], add=True)
            return 0

        lax.fori_loop(0, n_iter, body, 0)
        plsc.subcore_barrier()

        @pl.when(s == 0)
        def _flush():
            pltpu.sync_copy(shared_p, pp_hbm.at[c])

    return k(atom_feat, idx1d, zeros_p)[0]


def _graph_mlp_body(pp_ref, cc_ref, lat_ref, w3p_ref, w3l_ref, b3_ref,
                    w4_ref, b4_ref, mu_ref, lv_ref):
    pooled = pp_ref[0] + pp_ref[1]
    pooled = pooled / cc_ref[...]
    h2 = (jnp.dot(pooled, w3p_ref[...], preferred_element_type=jnp.float32)
          + jnp.dot(lat_ref[...], w3l_ref[...], preferred_element_type=jnp.float32)
          + b3_ref[...])
    h2 = jnp.maximum(h2, 0.0)
    params = (jnp.dot(h2, w4_ref[...], preferred_element_type=jnp.float32)
              + b4_ref[...])
    mu_ref[...] = params[:, :128]
    lv_ref[...] = params[:, 128:]


def _graph_mlp(pp, cc, lat9, w3p_t, w3l_t, b3r, w4_t, b4r):
    grid = G // BG
    return pl.pallas_call(
        _graph_mlp_body,
        grid=(grid,),
        in_specs=[
            pl.BlockSpec((NC, BG, 128), lambda i: (0, i, 0)),
            pl.BlockSpec((BG, 1), lambda i: (i, 0)),
            pl.BlockSpec((BG, 9), lambda i: (i, 0)),
            pl.BlockSpec((128, 128), lambda i: (0, 0)),
            pl.BlockSpec((9, 128), lambda i: (0, 0)),
            pl.BlockSpec((1, 128), lambda i: (0, 0)),
            pl.BlockSpec((128, 256), lambda i: (0, 0)),
            pl.BlockSpec((1, 256), lambda i: (0, 0)),
        ],
        out_specs=[pl.BlockSpec((BG, 128), lambda i: (i, 0)),
                   pl.BlockSpec((BG, 128), lambda i: (i, 0))],
        out_shape=[jax.ShapeDtypeStruct((G, 128), jnp.float32),
                   jax.ShapeDtypeStruct((G, 128), jnp.float32)],
        compiler_params=pltpu.CompilerParams(
            dimension_semantics=("parallel",)),
    )(pp, cc, lat9, w3p_t, w3l_t, b3r, w4_t, b4r)


def kernel(lattice, fracs, species, batch_indices, emb, W1, b1, W2, b2,
           W3, b3, W4, b4):
    idx1d = batch_indices.astype(jnp.int32)
    species_row = species.astype(jnp.int32).reshape(1, N)

    emb_p = jnp.zeros((128, 32), jnp.float32).at[:100].set(emb)
    w1e_t = W1[:, :32].T
    w1f_t = W1[:, 32:].T
    w2_t = W2.T
    w3p_t = W3[:, :128].T
    w3l_t = W3[:, 128:].T
    w4_t = W4.T

    atom_feat, counts = _atom_mlp(species_row, fracs, idx1d.reshape(1, N),
                                  emb_p, w1e_t, w1f_t,
                                  b1.reshape(1, 128), w2_t, b2.reshape(1, 128))
    cc = counts[:G]

    zeros_p = jnp.zeros((G, 128), jnp.float32)
    pp = _seg_sum_sc(atom_feat, idx1d, zeros_p)

    lat9 = lattice.reshape(G, 9)
    mu, logvar = _graph_mlp(pp, cc, lat9, w3p_t, w3l_t,
                            b3.reshape(1, 128), w4_t, b4.reshape(1, 256))
    return (mu, logvar)


# speedup vs baseline: 4.1904x; 1.2984x over previous
import functools

import jax
import jax.numpy as jnp
from jax import lax
from jax.experimental import pallas as pl
from jax.experimental.pallas import tpu as pltpu
from jax.experimental.pallas import tpu_sc as plsc

G = 10000
N = 320000
NC = 2
NS = 16
NW = NC * NS

B1 = 3200
PER_W = N // NW
CHUNK = 80
SW = 512
GPAD = 10240
BG = 2000


def _atom_mlp_body(sp_ref, fr_ref, idx_ref, emb_ref, w1e_ref, w1f_ref, b1_ref,
                   w2_ref, b2_ref, out_ref, cnt_ref):
    b = pl.program_id(0)

    @pl.when(b == 0)
    def _zero():
        cnt_ref[...] = jnp.zeros_like(cnt_ref)

    sp = sp_ref[...]
    rows = lax.broadcasted_iota(jnp.int32, (128, B1), 0)
    onehot_t = (rows == sp).astype(jnp.float32)
    e = lax.dot_general(onehot_t, emb_ref[...], (((0,), (0,)), ((), ())),
                        preferred_element_type=jnp.float32)
    h = (jnp.dot(e, w1e_ref[...], preferred_element_type=jnp.float32)
         + jnp.dot(fr_ref[...], w1f_ref[...], preferred_element_type=jnp.float32)
         + b1_ref[...])
    h = jnp.maximum(h, 0.0)
    out_ref[...] = (jnp.dot(h, w2_ref[...], preferred_element_type=jnp.float32)
                    + b2_ref[...])

    iv = idx_ref[...]
    lo = idx_ref[0, 0]
    hi = idx_ref[0, B1 - 1]
    base = (lo // SW) * SW
    n_strips = (hi - base) // SW + 1

    def strip(k, _):
        off = base + k * SW
        srows = off + lax.broadcasted_iota(jnp.int32, (SW, B1), 0)
        m = (srows == iv).astype(jnp.float32)
        cnt_ref[pl.ds(off, SW), :] += jnp.sum(m, axis=1, keepdims=True)
        return 0

    lax.fori_loop(0, n_strips, strip, 0)


def _atom_mlp(species_row, fracs, idx_row, emb_p, w1e_t, w1f_t, b1r, w2_t, b2r):
    grid = N // B1
    return pl.pallas_call(
        _atom_mlp_body,
        grid=(grid,),
        in_specs=[
            pl.BlockSpec((1, B1), lambda i: (0, i)),
            pl.BlockSpec((B1, 3), lambda i: (i, 0)),
            pl.BlockSpec((1, B1), lambda i: (0, i)),
            pl.BlockSpec((128, 32), lambda i: (0, 0)),
            pl.BlockSpec((32, 128), lambda i: (0, 0)),
            pl.BlockSpec((3, 128), lambda i: (0, 0)),
            pl.BlockSpec((1, 128), lambda i: (0, 0)),
            pl.BlockSpec((128, 128), lambda i: (0, 0)),
            pl.BlockSpec((1, 128), lambda i: (0, 0)),
        ],
        out_specs=[pl.BlockSpec((B1, 128), lambda i: (i, 0)),
                   pl.BlockSpec((GPAD, 1), lambda i: (0, 0))],
        out_shape=[jax.ShapeDtypeStruct((N, 128), jnp.float32),
                   jax.ShapeDtypeStruct((GPAD, 1), jnp.float32)],
        compiler_params=pltpu.CompilerParams(
            dimension_semantics=("arbitrary",)),
    )(species_row, fracs, idx_row, emb_p, w1e_t, w1f_t, b1r, w2_t, b2r)


def _seg_sum_sc(atom_feat, idx1d, zeros_p):
    mesh = plsc.VectorSubcoreMesh(core_axis_name="c", subcore_axis_name="s")
    n_iter = PER_W // CHUNK

    @functools.partial(
        pl.kernel, mesh=mesh,
        out_type=[jax.ShapeDtypeStruct((NC, G, 128), jnp.float32)],
        scratch_types=[
            pltpu.VMEM((CHUNK, 128), jnp.float32),
            pltpu.VMEM((CHUNK, 128), jnp.float32),
            pltpu.VMEM((CHUNK,), jnp.int32),
            pltpu.VMEM((CHUNK,), jnp.int32),
            pltpu.SemaphoreType.DMA,
            pltpu.SemaphoreType.DMA,
            pltpu.VMEM_SHARED((G, 128), jnp.float32),
        ],
    )
    def k(af_hbm, idx_hbm, zp_hbm, pp_hbm, rows0, rows1, iv0, iv1,
          sem0, sem1, shared_p):
        c = lax.axis_index("c")
        s = lax.axis_index("s")
        wid = c * NS + s
        base_w = wid * PER_W
        rows_b = (rows0, rows1)
        iv_b = (iv0, iv1)
        sem_b = (sem0, sem1)

        @pl.when(s == 0)
        def _init():
            pltpu.sync_copy(zp_hbm, shared_p)

        plsc.subcore_barrier()

        def fire(i, slot):
            rb = base_w + i * CHUNK
            pltpu.async_copy(idx_hbm.at[pl.ds(rb, CHUNK)], iv_b[slot],
                             sem_b[slot])
            pltpu.async_copy(af_hbm.at[pl.ds(rb, CHUNK)], rows_b[slot],
                             sem_b[slot])

        def drain_scatter(i, slot):
            rb = base_w + i * CHUNK
            pltpu.make_async_copy(idx_hbm.at[pl.ds(rb, CHUNK)], iv_b[slot],
                                  sem_b[slot]).wait()
            pltpu.make_async_copy(af_hbm.at[pl.ds(rb, CHUNK)], rows_b[slot],
                                  sem_b[slot]).wait()
            pltpu.sync_copy(rows_b[slot], shared_p.at[iv_b[slot]], add=True)

        fire(0, 0)

        def body(p, _):
            i0 = 2 * p
            fire(i0 + 1, 1)
            drain_scatter(i0, 0)
            fire(i0 + 2, 0)
            drain_scatter(i0 + 1, 1)
            return 0

        lax.fori_loop(0, (n_iter - 1) // 2, body, 0)
        drain_scatter(n_iter - 1, 0)

        plsc.subcore_barrier()

        @pl.when(s == 0)
        def _flush():
            pltpu.sync_copy(shared_p, pp_hbm.at[c])

    return k(atom_feat, idx1d, zeros_p)[0]


def _graph_mlp_body(pp_ref, cc_ref, lat_ref, w3p_ref, w3l_ref, b3_ref,
                    w4_ref, b4_ref, mu_ref, lv_ref):
    pooled = pp_ref[0] + pp_ref[1]
    pooled = pooled / cc_ref[...]
    h2 = (jnp.dot(pooled, w3p_ref[...], preferred_element_type=jnp.float32)
          + jnp.dot(lat_ref[...], w3l_ref[...], preferred_element_type=jnp.float32)
          + b3_ref[...])
    h2 = jnp.maximum(h2, 0.0)
    params = (jnp.dot(h2, w4_ref[...], preferred_element_type=jnp.float32)
              + b4_ref[...])
    mu_ref[...] = params[:, :128]
    lv_ref[...] = params[:, 128:]


def _graph_mlp(pp, cc, lat9, w3p_t, w3l_t, b3r, w4_t, b4r):
    grid = G // BG
    return pl.pallas_call(
        _graph_mlp_body,
        grid=(grid,),
        in_specs=[
            pl.BlockSpec((NC, BG, 128), lambda i: (0, i, 0)),
            pl.BlockSpec((BG, 1), lambda i: (i, 0)),
            pl.BlockSpec((BG, 9), lambda i: (i, 0)),
            pl.BlockSpec((128, 128), lambda i: (0, 0)),
            pl.BlockSpec((9, 128), lambda i: (0, 0)),
            pl.BlockSpec((1, 128), lambda i: (0, 0)),
            pl.BlockSpec((128, 256), lambda i: (0, 0)),
            pl.BlockSpec((1, 256), lambda i: (0, 0)),
        ],
        out_specs=[pl.BlockSpec((BG, 128), lambda i: (i, 0)),
                   pl.BlockSpec((BG, 128), lambda i: (i, 0))],
        out_shape=[jax.ShapeDtypeStruct((G, 128), jnp.float32),
                   jax.ShapeDtypeStruct((G, 128), jnp.float32)],
        compiler_params=pltpu.CompilerParams(
            dimension_semantics=("parallel",)),
    )(pp, cc, lat9, w3p_t, w3l_t, b3r, w4_t, b4r)


def kernel(lattice, fracs, species, batch_indices, emb, W1, b1, W2, b2,
           W3, b3, W4, b4):
    idx1d = batch_indices.astype(jnp.int32)
    species_row = species.astype(jnp.int32).reshape(1, N)

    emb_p = jnp.zeros((128, 32), jnp.float32).at[:100].set(emb)
    w1e_t = W1[:, :32].T
    w1f_t = W1[:, 32:].T
    w2_t = W2.T
    w3p_t = W3[:, :128].T
    w3l_t = W3[:, 128:].T
    w4_t = W4.T

    atom_feat, counts = _atom_mlp(species_row, fracs, idx1d.reshape(1, N),
                                  emb_p, w1e_t, w1f_t,
                                  b1.reshape(1, 128), w2_t, b2.reshape(1, 128))
    cc = counts[:G]

    zeros_p = jnp.zeros((G, 128), jnp.float32)
    pp = _seg_sum_sc(atom_feat, idx1d, zeros_p)

    lat9 = lattice.reshape(G, 9)
    mu, logvar = _graph_mlp(pp, cc, lat9, w3p_t, w3l_t,
                            b3.reshape(1, 128), w4_t, b4.reshape(1, 256))
    return (mu, logvar)


# speedup vs baseline: 4.9136x; 1.1726x over previous
import functools

import jax
import jax.numpy as jnp
from jax import lax
from jax.experimental import pallas as pl
from jax.experimental.pallas import tpu as pltpu
from jax.experimental.pallas import tpu_sc as plsc

G = 10000
N = 320000
NC = 2
NS = 16
NW = NC * NS

B1 = 3200
PER_W = N // NW
CHUNK = 80
SW = 512
GPAD = 10240
BG = 2000


def _atom_mlp_body(sp_ref, f0_ref, f1_ref, f2_ref, idx_ref, m1_ref, b1_ref,
                   w2_ref, b2_ref, out_ref, cnt_ref):
    b = pl.program_id(0)

    @pl.when(b == 0)
    def _zero():
        cnt_ref[...] = jnp.zeros_like(cnt_ref)

    sp = sp_ref[...]
    rows = lax.broadcasted_iota(jnp.int32, (128, B1), 0)
    onehot_t = (rows == sp).astype(jnp.float32)
    a = jnp.concatenate(
        [onehot_t, f0_ref[...], f1_ref[...], f2_ref[...]], axis=0)
    h = lax.dot_general(a, m1_ref[...], (((0,), (0,)), ((), ())),
                        preferred_element_type=jnp.float32) + b1_ref[...]
    h = jnp.maximum(h, 0.0)
    out_ref[...] = (jnp.dot(h, w2_ref[...], preferred_element_type=jnp.float32)
                    + b2_ref[...])

    iv = idx_ref[...]
    lo = idx_ref[0, 0]
    hi = idx_ref[0, B1 - 1]
    base = (lo // SW) * SW
    n_strips = (hi - base) // SW + 1

    def strip(k, _):
        off = base + k * SW
        srows = off + lax.broadcasted_iota(jnp.int32, (SW, B1), 0)
        m = (srows == iv).astype(jnp.float32)
        cnt_ref[pl.ds(off, SW), :] += jnp.sum(m, axis=1, keepdims=True)
        return 0

    lax.fori_loop(0, n_strips, strip, 0)


def _atom_mlp(species_row, f0, f1, f2, idx_row, m1_aug, b1r, w2_t, b2r):
    grid = N // B1
    return pl.pallas_call(
        _atom_mlp_body,
        grid=(grid,),
        in_specs=[
            pl.BlockSpec((1, B1), lambda i: (0, i)),
            pl.BlockSpec((1, B1), lambda i: (0, i)),
            pl.BlockSpec((1, B1), lambda i: (0, i)),
            pl.BlockSpec((1, B1), lambda i: (0, i)),
            pl.BlockSpec((1, B1), lambda i: (0, i)),
            pl.BlockSpec((131, 128), lambda i: (0, 0)),
            pl.BlockSpec((1, 128), lambda i: (0, 0)),
            pl.BlockSpec((128, 128), lambda i: (0, 0)),
            pl.BlockSpec((1, 128), lambda i: (0, 0)),
        ],
        out_specs=[pl.BlockSpec((B1, 128), lambda i: (i, 0)),
                   pl.BlockSpec((GPAD, 1), lambda i: (0, 0))],
        out_shape=[jax.ShapeDtypeStruct((N, 128), jnp.float32),
                   jax.ShapeDtypeStruct((GPAD, 1), jnp.float32)],
        compiler_params=pltpu.CompilerParams(
            dimension_semantics=("arbitrary",)),
    )(species_row, f0, f1, f2, idx_row, m1_aug, b1r, w2_t, b2r)


def _seg_sum_sc(atom_feat, idx1d, zeros_p):
    mesh = plsc.VectorSubcoreMesh(core_axis_name="c", subcore_axis_name="s")
    n_iter = PER_W // CHUNK

    @functools.partial(
        pl.kernel, mesh=mesh,
        out_type=[jax.ShapeDtypeStruct((NC, G, 128), jnp.float32)],
        scratch_types=[
            pltpu.VMEM((CHUNK, 128), jnp.float32),
            pltpu.VMEM((CHUNK, 128), jnp.float32),
            pltpu.VMEM((CHUNK,), jnp.int32),
            pltpu.VMEM((CHUNK,), jnp.int32),
            pltpu.SemaphoreType.DMA,
            pltpu.SemaphoreType.DMA,
            pltpu.VMEM_SHARED((G, 128), jnp.float32),
        ],
    )
    def k(af_hbm, idx_hbm, zp_hbm, pp_hbm, rows0, rows1, iv0, iv1,
          sem0, sem1, shared_p):
        c = lax.axis_index("c")
        s = lax.axis_index("s")
        wid = c * NS + s
        base_w = wid * PER_W
        rows_b = (rows0, rows1)
        iv_b = (iv0, iv1)
        sem_b = (sem0, sem1)

        @pl.when(s == 0)
        def _init():
            pltpu.sync_copy(zp_hbm, shared_p)

        plsc.subcore_barrier()

        def fire(i, slot):
            rb = base_w + i * CHUNK
            pltpu.async_copy(idx_hbm.at[pl.ds(rb, CHUNK)], iv_b[slot],
                             sem_b[slot])
            pltpu.async_copy(af_hbm.at[pl.ds(rb, CHUNK)], rows_b[slot],
                             sem_b[slot])

        def drain_scatter(i, slot):
            rb = base_w + i * CHUNK
            pltpu.make_async_copy(idx_hbm.at[pl.ds(rb, CHUNK)], iv_b[slot],
                                  sem_b[slot]).wait()
            pltpu.make_async_copy(af_hbm.at[pl.ds(rb, CHUNK)], rows_b[slot],
                                  sem_b[slot]).wait()
            pltpu.sync_copy(rows_b[slot], shared_p.at[iv_b[slot]], add=True)

        fire(0, 0)

        def body(p, _):
            i0 = 2 * p
            fire(i0 + 1, 1)
            drain_scatter(i0, 0)
            fire(i0 + 2, 0)
            drain_scatter(i0 + 1, 1)
            return 0

        lax.fori_loop(0, (n_iter - 1) // 2, body, 0)
        drain_scatter(n_iter - 1, 0)

        plsc.subcore_barrier()

        @pl.when(s == 0)
        def _flush():
            pltpu.sync_copy(shared_p, pp_hbm.at[c])

    return k(atom_feat, idx1d, zeros_p)[0]


def _graph_mlp_body(pp_ref, cc_ref, lat_ref, w3p_ref, w3l_ref, b3_ref,
                    w4_ref, b4_ref, mu_ref, lv_ref):
    pooled = pp_ref[0] + pp_ref[1]
    pooled = pooled / cc_ref[...]
    h2 = (jnp.dot(pooled, w3p_ref[...], preferred_element_type=jnp.float32)
          + jnp.dot(lat_ref[...], w3l_ref[...], preferred_element_type=jnp.float32)
          + b3_ref[...])
    h2 = jnp.maximum(h2, 0.0)
    params = (jnp.dot(h2, w4_ref[...], preferred_element_type=jnp.float32)
              + b4_ref[...])
    mu_ref[...] = params[:, :128]
    lv_ref[...] = params[:, 128:]


def _graph_mlp(pp, cc, lat9, w3p_t, w3l_t, b3r, w4_t, b4r):
    grid = G // BG
    return pl.pallas_call(
        _graph_mlp_body,
        grid=(grid,),
        in_specs=[
            pl.BlockSpec((NC, BG, 128), lambda i: (0, i, 0)),
            pl.BlockSpec((BG, 1), lambda i: (i, 0)),
            pl.BlockSpec((BG, 9), lambda i: (i, 0)),
            pl.BlockSpec((128, 128), lambda i: (0, 0)),
            pl.BlockSpec((9, 128), lambda i: (0, 0)),
            pl.BlockSpec((1, 128), lambda i: (0, 0)),
            pl.BlockSpec((128, 256), lambda i: (0, 0)),
            pl.BlockSpec((1, 256), lambda i: (0, 0)),
        ],
        out_specs=[pl.BlockSpec((BG, 128), lambda i: (i, 0)),
                   pl.BlockSpec((BG, 128), lambda i: (i, 0))],
        out_shape=[jax.ShapeDtypeStruct((G, 128), jnp.float32),
                   jax.ShapeDtypeStruct((G, 128), jnp.float32)],
        compiler_params=pltpu.CompilerParams(
            dimension_semantics=("parallel",)),
    )(pp, cc, lat9, w3p_t, w3l_t, b3r, w4_t, b4r)


def kernel(lattice, fracs, species, batch_indices, emb, W1, b1, W2, b2,
           W3, b3, W4, b4):
    idx1d = batch_indices.astype(jnp.int32)
    species_row = species.astype(jnp.int32).reshape(1, N)

    emb_p = jnp.zeros((128, 32), jnp.float32).at[:100].set(emb)
    m1_aug = jnp.concatenate([emb_p @ W1[:, :32].T, W1[:, 32:].T], axis=0)
    w2_t = W2.T
    f0 = fracs[:, 0].reshape(1, N)
    f1 = fracs[:, 1].reshape(1, N)
    f2 = fracs[:, 2].reshape(1, N)
    w3p_t = W3[:, :128].T
    w3l_t = W3[:, 128:].T
    w4_t = W4.T

    atom_feat, counts = _atom_mlp(species_row, f0, f1, f2, idx1d.reshape(1, N),
                                  m1_aug, b1.reshape(1, 128), w2_t,
                                  b2.reshape(1, 128))
    cc = counts[:G]

    zeros_p = jnp.zeros((G, 128), jnp.float32)
    pp = _seg_sum_sc(atom_feat, idx1d, zeros_p)

    lat9 = lattice.reshape(G, 9)
    mu, logvar = _graph_mlp(pp, cc, lat9, w3p_t, w3l_t,
                            b3.reshape(1, 128), w4_t, b4.reshape(1, 256))
    return (mu, logvar)


# speedup vs baseline: 5.5374x; 1.1269x over previous
import functools

import jax
import jax.numpy as jnp
from jax import lax
from jax.experimental import pallas as pl
from jax.experimental.pallas import tpu as pltpu
from jax.experimental.pallas import tpu_sc as plsc

G = 10000
N = 320000
NC = 2
NS = 16
NW = NC * NS

B1 = 3200
PER_W = N // NW
CHUNK = 80
SW = 256
GPAD = 10240
BG = 2000


def _atom_mlp_body(sp_ref, f0_ref, f1_ref, f2_ref, idx_ref, m1_ref, b1_ref,
                   w2_ref, b2_ref, out_ref, cnt_ref):
    b = pl.program_id(0)

    @pl.when(b == 0)
    def _zero():
        cnt_ref[...] = jnp.zeros_like(cnt_ref)

    sp = sp_ref[...]
    rows = lax.broadcasted_iota(jnp.int32, (128, B1), 0)
    onehot_t = (rows == sp).astype(jnp.float32)
    a = jnp.concatenate(
        [onehot_t, f0_ref[...], f1_ref[...], f2_ref[...]], axis=0)
    h = lax.dot_general(a.astype(jnp.bfloat16), m1_ref[...],
                        (((0,), (0,)), ((), ())),
                        preferred_element_type=jnp.float32) + b1_ref[...]
    h = jnp.maximum(h, 0.0)
    out_ref[...] = (jnp.dot(h.astype(jnp.bfloat16), w2_ref[...],
                            preferred_element_type=jnp.float32)
                    + b2_ref[...])

    iv = idx_ref[...]
    lo = idx_ref[0, 0]
    hi = idx_ref[0, B1 - 1]
    base = (lo // SW) * SW
    n_strips = (hi - base) // SW + 1

    def strip(k, _):
        off = base + k * SW
        srows = off + lax.broadcasted_iota(jnp.int32, (SW, B1), 0)
        m = (srows == iv).astype(jnp.float32)
        cnt_ref[pl.ds(off, SW), :] += jnp.sum(m, axis=1, keepdims=True)
        return 0

    lax.fori_loop(0, n_strips, strip, 0)


def _atom_mlp(species_row, f0, f1, f2, idx_row, m1_aug, b1r, w2_t, b2r):
    grid = N // B1
    return pl.pallas_call(
        _atom_mlp_body,
        grid=(grid,),
        in_specs=[
            pl.BlockSpec((1, B1), lambda i: (0, i)),
            pl.BlockSpec((1, B1), lambda i: (0, i)),
            pl.BlockSpec((1, B1), lambda i: (0, i)),
            pl.BlockSpec((1, B1), lambda i: (0, i)),
            pl.BlockSpec((1, B1), lambda i: (0, i)),
            pl.BlockSpec((131, 128), lambda i: (0, 0)),
            pl.BlockSpec((1, 128), lambda i: (0, 0)),
            pl.BlockSpec((128, 128), lambda i: (0, 0)),
            pl.BlockSpec((1, 128), lambda i: (0, 0)),
        ],
        out_specs=[pl.BlockSpec((B1, 128), lambda i: (i, 0)),
                   pl.BlockSpec((GPAD, 1), lambda i: (0, 0))],
        out_shape=[jax.ShapeDtypeStruct((N, 128), jnp.float32),
                   jax.ShapeDtypeStruct((GPAD, 1), jnp.float32)],
        compiler_params=pltpu.CompilerParams(
            dimension_semantics=("arbitrary",)),
    )(species_row, f0, f1, f2, idx_row, m1_aug, b1r, w2_t, b2r)


def _seg_sum_sc(atom_feat, idx1d, zeros_p):
    mesh = plsc.VectorSubcoreMesh(core_axis_name="c", subcore_axis_name="s")
    n_iter = PER_W // CHUNK

    @functools.partial(
        pl.kernel, mesh=mesh,
        out_type=[jax.ShapeDtypeStruct((NC, G, 128), jnp.float32)],
        scratch_types=[
            pltpu.VMEM((CHUNK, 128), jnp.float32),
            pltpu.VMEM((CHUNK, 128), jnp.float32),
            pltpu.VMEM((CHUNK,), jnp.int32),
            pltpu.VMEM((CHUNK,), jnp.int32),
            pltpu.SemaphoreType.DMA,
            pltpu.SemaphoreType.DMA,
            pltpu.VMEM_SHARED((G, 128), jnp.float32),
        ],
    )
    def k(af_hbm, idx_hbm, zp_hbm, pp_hbm, rows0, rows1, iv0, iv1,
          sem0, sem1, shared_p):
        c = lax.axis_index("c")
        s = lax.axis_index("s")
        wid = c * NS + s
        base_w = wid * PER_W
        rows_b = (rows0, rows1)
        iv_b = (iv0, iv1)
        sem_b = (sem0, sem1)

        @pl.when(s == 0)
        def _init():
            pltpu.sync_copy(zp_hbm, shared_p)

        plsc.subcore_barrier()

        def fire(i, slot):
            rb = base_w + i * CHUNK
            pltpu.async_copy(idx_hbm.at[pl.ds(rb, CHUNK)], iv_b[slot],
                             sem_b[slot])
            pltpu.async_copy(af_hbm.at[pl.ds(rb, CHUNK)], rows_b[slot],
                             sem_b[slot])

        def drain_scatter(i, slot):
            rb = base_w + i * CHUNK
            pltpu.make_async_copy(idx_hbm.at[pl.ds(rb, CHUNK)], iv_b[slot],
                                  sem_b[slot]).wait()
            pltpu.make_async_copy(af_hbm.at[pl.ds(rb, CHUNK)], rows_b[slot],
                                  sem_b[slot]).wait()
            pltpu.sync_copy(rows_b[slot], shared_p.at[iv_b[slot]], add=True)

        fire(0, 0)

        def body(p, _):
            i0 = 2 * p
            fire(i0 + 1, 1)
            drain_scatter(i0, 0)
            fire(i0 + 2, 0)
            drain_scatter(i0 + 1, 1)
            return 0

        lax.fori_loop(0, (n_iter - 1) // 2, body, 0)
        drain_scatter(n_iter - 1, 0)

        plsc.subcore_barrier()

        @pl.when(s == 0)
        def _flush():
            pltpu.sync_copy(shared_p, pp_hbm.at[c])

    return k(atom_feat, idx1d, zeros_p)[0]


def _graph_mlp_body(pp_ref, cc_ref, lat_ref, w3p_ref, w3l_ref, b3_ref,
                    w4_ref, b4_ref, mu_ref, lv_ref):
    pooled = pp_ref[0] + pp_ref[1]
    pooled = pooled / cc_ref[...]
    h2 = (jnp.dot(pooled, w3p_ref[...], preferred_element_type=jnp.float32)
          + jnp.dot(lat_ref[...], w3l_ref[...], preferred_element_type=jnp.float32)
          + b3_ref[...])
    h2 = jnp.maximum(h2, 0.0)
    params = (jnp.dot(h2, w4_ref[...], preferred_element_type=jnp.float32)
              + b4_ref[...])
    mu_ref[...] = params[:, :128]
    lv_ref[...] = params[:, 128:]


def _graph_mlp(pp, cc, lat9, w3p_t, w3l_t, b3r, w4_t, b4r):
    grid = G // BG
    return pl.pallas_call(
        _graph_mlp_body,
        grid=(grid,),
        in_specs=[
            pl.BlockSpec((NC, BG, 128), lambda i: (0, i, 0)),
            pl.BlockSpec((BG, 1), lambda i: (i, 0)),
            pl.BlockSpec((BG, 9), lambda i: (i, 0)),
            pl.BlockSpec((128, 128), lambda i: (0, 0)),
            pl.BlockSpec((9, 128), lambda i: (0, 0)),
            pl.BlockSpec((1, 128), lambda i: (0, 0)),
            pl.BlockSpec((128, 256), lambda i: (0, 0)),
            pl.BlockSpec((1, 256), lambda i: (0, 0)),
        ],
        out_specs=[pl.BlockSpec((BG, 128), lambda i: (i, 0)),
                   pl.BlockSpec((BG, 128), lambda i: (i, 0))],
        out_shape=[jax.ShapeDtypeStruct((G, 128), jnp.float32),
                   jax.ShapeDtypeStruct((G, 128), jnp.float32)],
        compiler_params=pltpu.CompilerParams(
            dimension_semantics=("parallel",)),
    )(pp, cc, lat9, w3p_t, w3l_t, b3r, w4_t, b4r)


def kernel(lattice, fracs, species, batch_indices, emb, W1, b1, W2, b2,
           W3, b3, W4, b4):
    idx1d = batch_indices.astype(jnp.int32)
    species_row = species.astype(jnp.int32).reshape(1, N)

    emb_p = jnp.zeros((128, 32), jnp.float32).at[:100].set(emb)
    m1_aug = jnp.concatenate(
        [emb_p @ W1[:, :32].T, W1[:, 32:].T], axis=0).astype(jnp.bfloat16)
    w2_t = W2.T.astype(jnp.bfloat16)
    f0 = fracs[:, 0].reshape(1, N)
    f1 = fracs[:, 1].reshape(1, N)
    f2 = fracs[:, 2].reshape(1, N)
    w3p_t = W3[:, :128].T
    w3l_t = W3[:, 128:].T
    w4_t = W4.T

    atom_feat, counts = _atom_mlp(species_row, f0, f1, f2, idx1d.reshape(1, N),
                                  m1_aug, b1.reshape(1, 128), w2_t,
                                  b2.reshape(1, 128))
    cc = counts[:G]

    zeros_p = jnp.zeros((G, 128), jnp.float32)
    pp = _seg_sum_sc(atom_feat, idx1d, zeros_p)

    lat9 = lattice.reshape(G, 9)
    mu, logvar = _graph_mlp(pp, cc, lat9, w3p_t, w3l_t,
                            b3.reshape(1, 128), w4_t, b4.reshape(1, 256))
    return (mu, logvar)
